# baseline (speedup 1.0000x reference)
import dataclasses
import functools

import jax
import jax.numpy as jnp
from jax import lax
from jax.experimental import pallas as pl
from jax.experimental.pallas import tpu as pltpu
from jax.experimental.pallas import tpu_sc as plsc

T = 2048
H = 1024
I = 1408
E = 8
K = 2
B = 1024
CN = 176
NB = (K * T) // B + E
NPAD = NB * B

NC = 2
NS = 16
NW = NC * NS
TPW = T // NW
DCH = 32
CCH = 16
@functools.cache
def _sc_mesh():
    return plsc.VectorSubcoreMesh(core_axis_name="c", subcore_axis_name="s")

_SC_PARAMS = pltpu.CompilerParams()
if "needs_layout_passes" in pltpu.CompilerParams.__dataclass_fields__:
    _SC_PARAMS = dataclasses.replace(_SC_PARAMS, needs_layout_passes=False)


def _dispatch_body(hid_hbm, s1_hbm, s2_hbm, xs_hbm, buf0, buf1,
                   idx1, idx2, rs0, rs1, ws0, ws1):
    wid = lax.axis_index("s") * NC + lax.axis_index("c")
    base = wid * TPW
    pltpu.sync_copy(s1_hbm.at[wid], idx1)
    pltpu.sync_copy(s2_hbm.at[wid], idx2)
    bufs = (buf0, buf1)
    rsems = (rs0, rs1)
    wsems = (ws0, ws1)
    nsub = TPW // DCH

    def read(j):
        p = j % 2
        return pltpu.async_copy(hid_hbm.at[pl.ds(base + j * DCH, DCH)],
                                bufs[p], rsems[p])

    pend = {0: read(0), 1: read(1)}
    pw = {}
    for j in range(nsub):
        p = j % 2
        pend.pop(j).wait()
        c1 = pltpu.async_copy(bufs[p], xs_hbm.at[idx1.at[j]], wsems[p])
        c2 = pltpu.async_copy(bufs[p], xs_hbm.at[idx2.at[j]], wsems[p])
        pw[j] = (c1, c2)
        if j + 2 < nsub:
            for c in pw.pop(j):
                c.wait()
            pend[j + 2] = read(j + 2)
    for cs in pw.values():
        for c in cs:
            c.wait()


def _sc_dispatch(hidden_states, s1, s2):
    s1r = s1.reshape(NW, TPW // DCH, DCH)
    s2r = s2.reshape(NW, TPW // DCH, DCH)
    k = functools.partial(
        pl.kernel,
        mesh=_sc_mesh(),
        out_type=jax.ShapeDtypeStruct((NPAD, H), jnp.float32),
        scratch_types=[
            pltpu.VMEM((DCH, H), jnp.float32),
            pltpu.VMEM((DCH, H), jnp.float32),
            pltpu.VMEM((TPW // DCH, DCH), jnp.int32),
            pltpu.VMEM((TPW // DCH, DCH), jnp.int32),
            pltpu.SemaphoreType.DMA,
            pltpu.SemaphoreType.DMA,
            pltpu.SemaphoreType.DMA,
            pltpu.SemaphoreType.DMA,
        ],
    )(_dispatch_body)
    return k(hidden_states, s1r, s2r)


_NSUB = TPW // CCH


def _combine_body(ys_hbm, s1_hbm, s2_hbm, rw1_hbm, rw2_hbm, out_hbm,
                  a0, a1, b0, b1, o0, o1, idx1, idx2, rw1v, rw2v,
                  ga0, ga1, gb0, gb1, os0, os1):
    wid = lax.axis_index("s") * NC + lax.axis_index("c")
    base = wid * TPW
    pltpu.sync_copy(s1_hbm.at[wid], idx1)
    pltpu.sync_copy(s2_hbm.at[wid], idx2)
    pltpu.sync_copy(rw1_hbm.at[wid], rw1v)
    pltpu.sync_copy(rw2_hbm.at[wid], rw2v)
    ab = (a0, a1)
    bb = (b0, b1)
    ob = (o0, o1)
    gas = (ga0, ga1)
    gbs = (gb0, gb1)
    oss = (os0, os1)

    def gather(j):
        p = j % 2
        c1 = pltpu.async_copy(ys_hbm.at[idx1.at[j]], ab[p], gas[p])
        c2 = pltpu.async_copy(ys_hbm.at[idx2.at[j]], bb[p], gbs[p])
        return c1, c2

    pend = {0: gather(0), 1: gather(1)}
    pstore = {}
    for j in range(_NSUB):
        p = j % 2
        c1, c2 = pend.pop(j)
        c1.wait()
        c2.wait()
        if j - 2 in pstore:
            pstore.pop(j - 2).wait()

        @pl.loop(0, CCH)
        def _(r):
            g = jnp.full((16,), j * CCH + r, jnp.int32)
            w1 = plsc.load_gather(rw1v, [g])
            w2 = plsc.load_gather(rw2v, [g])
            for c in range(0, H, 16):
                ob[p][r, pl.ds(c, 16)] = (ab[p][r, pl.ds(c, 16)] * w1
                                          + bb[p][r, pl.ds(c, 16)] * w2)

        pstore[j] = pltpu.async_copy(
            ob[p], out_hbm.at[pl.ds(base + j * CCH, CCH)], oss[p])
        if j + 2 < _NSUB:
            pend[j + 2] = gather(j + 2)
    for cp in pstore.values():
        cp.wait()


def _sc_combine(ys, s1, s2, rw1, rw2):
    s1r = s1.reshape(NW, _NSUB, CCH)
    s2r = s2.reshape(NW, _NSUB, CCH)
    rw1r = rw1.reshape(NW, TPW)
    rw2r = rw2.reshape(NW, TPW)
    k = functools.partial(
        pl.kernel,
        mesh=_sc_mesh(),
        out_type=jax.ShapeDtypeStruct((T, H), jnp.float32),
        scratch_types=(
            [pltpu.VMEM((CCH, H), jnp.float32) for _ in range(6)]
            + [pltpu.VMEM((_NSUB, CCH), jnp.int32) for _ in range(2)]
            + [pltpu.VMEM((TPW,), jnp.float32) for _ in range(2)]
            + [pltpu.SemaphoreType.DMA for _ in range(6)]
        ),
        compiler_params=_SC_PARAMS,
    )(_combine_body)
    return k(ys, s1r, s2r, rw1r, rw2r)


def _meta_body(lt_ref, s1_ref, s2_ref, rw1_ref, rw2_ref, be_ref, nbt_ref):
    lt = lt_ref[...]
    NEG = jnp.float32(-1e30)
    m1 = jnp.full((1, T), NEG, jnp.float32)
    i1 = jnp.zeros((1, T), jnp.int32)
    for e in range(E):
        ce = lt[e:e + 1, :]
        upd = ce > m1
        i1 = jnp.where(upd, e, i1)
        m1 = jnp.where(upd, ce, m1)
    m2 = jnp.full((1, T), NEG, jnp.float32)
    i2 = jnp.zeros((1, T), jnp.int32)
    for e in range(E):
        ce = jnp.where(i1 == e, NEG, lt[e:e + 1, :])
        upd = ce > m2
        i2 = jnp.where(upd, e, i2)
        m2 = jnp.where(upd, ce, m2)
    rw1_ref[...] = jax.nn.sigmoid(m1 - m2)
    rw2_ref[...] = jax.nn.sigmoid(m2 - m1)

    ids = jnp.concatenate([i1, i2], axis=1)
    eio = jax.lax.broadcasted_iota(jnp.int32, (E, K * T), 0)
    oh = (ids == eio).astype(jnp.float32)
    lio = jax.lax.broadcasted_iota(jnp.int32, (E, K * T), 1)
    c = oh
    sh = 1
    while sh < K * T:
        c = c + jnp.where(lio >= sh, pltpu.roll(c, sh, axis=1), 0.0)
        sh *= 2
    counts = c[:, K * T - 1:K * T]
    rank = jnp.sum(oh * (c - 1.0), axis=0, keepdims=True)
    nblk = jnp.ceil(counts / B)
    sio = jax.lax.broadcasted_iota(jnp.int32, (E, 1), 0)
    cb = nblk
    for s in (1, 2, 4):
        cb = cb + jnp.where(sio >= s, pltpu.roll(cb, s, axis=0), 0.0)
    row_off = (cb - nblk) * B
    slots = (rank + jnp.sum(oh * row_off, axis=0, keepdims=True)).astype(jnp.int32)
    s1_ref[...] = slots[:, :T]
    s2_ref[...] = slots[:, T:]
    bio = jax.lax.broadcasted_iota(jnp.int32, (E, NB), 1).astype(jnp.float32)
    becnt = jnp.sum((bio >= cb).astype(jnp.int32), axis=0, keepdims=True)
    last_e = jnp.max(jnp.where(nblk > 0.0, sio, 0), axis=0, keepdims=True)
    be_ref[...] = jnp.minimum(becnt, last_e)
    nbt_ref[...] = cb[E - 1:E, :].astype(jnp.int32)


def _tc_metadata(router_logits_t):
    out_shapes = [
        jax.ShapeDtypeStruct((1, T), jnp.int32),
        jax.ShapeDtypeStruct((1, T), jnp.int32),
        jax.ShapeDtypeStruct((1, T), jnp.float32),
        jax.ShapeDtypeStruct((1, T), jnp.float32),
        jax.ShapeDtypeStruct((1, NB), jnp.int32),
        jax.ShapeDtypeStruct((1, 1), jnp.int32),
    ]
    return pl.pallas_call(_meta_body, out_shape=out_shapes)(router_logits_t)


def _ffn_block(be_ref, nbt_ref, xs_ref, w13_ref, w2_ref, ys_ref,
               w13_bf, w2_bf, act_bf):
    b = pl.program_id(0)

    @pl.when(b < nbt_ref[0])
    def _():
        prev = be_ref[jnp.maximum(b - 1, 0)]

        @pl.when((b == 0) | (be_ref[b] != prev))
        def _():
            w13_bf[...] = w13_ref[0].astype(jnp.bfloat16)
            w2_bf[...] = w2_ref[0].astype(jnp.bfloat16)

        x = xs_ref[...].astype(jnp.bfloat16)
        for c in range(0, I, CN):
            g = jax.lax.dot_general(x, w13_bf[c:c + CN, :],
                                    (((1,), (1,)), ((), ())),
                                    preferred_element_type=jnp.float32)
            u = jax.lax.dot_general(x, w13_bf[I + c:I + c + CN, :],
                                    (((1,), (1,)), ((), ())),
                                    preferred_element_type=jnp.float32)
            act_bf[:, c:c + CN] = (g * jax.nn.sigmoid(g) * u).astype(jnp.bfloat16)
        y = jax.lax.dot_general(act_bf[...], w2_bf[...], (((1,), (1,)), ((), ())),
                                preferred_element_type=jnp.float32)
        ys_ref[...] = y


def _grouped_ffn(block_expert, nb_total, xs, w13, w2):
    grid_spec = pltpu.PrefetchScalarGridSpec(
        num_scalar_prefetch=2,
        grid=(NB,),
        in_specs=[
            pl.BlockSpec((B, H),
                         lambda b, be, nbt: (jnp.minimum(b, nbt[0] - 1), 0)),
            pl.BlockSpec((1, 2 * I, H), lambda b, be, nbt: (be[b], 0, 0)),
            pl.BlockSpec((1, H, I), lambda b, be, nbt: (be[b], 0, 0)),
        ],
        out_specs=pl.BlockSpec((B, H),
                               lambda b, be, nbt: (jnp.minimum(b, nbt[0] - 1), 0)),
        scratch_shapes=[
            pltpu.VMEM((2 * I, H), jnp.bfloat16),
            pltpu.VMEM((H, I), jnp.bfloat16),
            pltpu.VMEM((B, I), jnp.bfloat16),
        ],
    )
    return pl.pallas_call(
        _ffn_block,
        grid_spec=grid_spec,
        out_shape=jax.ShapeDtypeStruct((NPAD, H), jnp.float32),
        compiler_params=pltpu.CompilerParams(
            vmem_limit_bytes=64 * 1024 * 1024),
    )(block_expert, nb_total, xs, w13, w2)


def kernel(hidden_states, router_logits, w13, w2):
    s1o, s2o, rw1o, rw2o, beo, nbto = _tc_metadata(
        router_logits.astype(jnp.float32).T)
    s1 = s1o.reshape(T)
    s2 = s2o.reshape(T)
    rw1 = rw1o.reshape(T)
    rw2 = rw2o.reshape(T)
    block_expert = beo.reshape(NB)
    nb_total = nbto.reshape(1)

    xs = _sc_dispatch(hidden_states, s1, s2)

    ys = _grouped_ffn(block_expert, nb_total, xs, w13, w2)

    return _sc_combine(ys, s1, s2, rw1, rw2)


# speedup vs baseline: 1.1916x; 1.1916x over previous
import dataclasses
import functools

import jax
import jax.numpy as jnp
from jax import lax
from jax.experimental import pallas as pl
from jax.experimental.pallas import tpu as pltpu
from jax.experimental.pallas import tpu_sc as plsc

T = 2048
H = 1024
I = 1408
E = 8
K = 2
B = 512
NB = (K * T) // B + E
NPAD = NB * B

NC = 2
NS = 16
NW = NC * NS
TPW = T // NW
DCH = 32
CCH = 16
@functools.cache
def _sc_mesh():
    return plsc.VectorSubcoreMesh(core_axis_name="c", subcore_axis_name="s")

_SC_PARAMS = pltpu.CompilerParams()
if "needs_layout_passes" in pltpu.CompilerParams.__dataclass_fields__:
    _SC_PARAMS = dataclasses.replace(_SC_PARAMS, needs_layout_passes=False)


def _dispatch_body(hid_hbm, s1_hbm, s2_hbm, xs_hbm, buf0, buf1,
                   idx1, idx2, rs0, rs1, ws0, ws1):
    wid = lax.axis_index("s") * NC + lax.axis_index("c")
    base = wid * TPW
    pltpu.sync_copy(s1_hbm.at[wid], idx1)
    pltpu.sync_copy(s2_hbm.at[wid], idx2)
    bufs = (buf0, buf1)
    rsems = (rs0, rs1)
    wsems = (ws0, ws1)
    nsub = TPW // DCH

    def read(j):
        p = j % 2
        return pltpu.async_copy(hid_hbm.at[pl.ds(base + j * DCH, DCH)],
                                bufs[p], rsems[p])

    pend = {0: read(0), 1: read(1)}
    pw = {}
    for j in range(nsub):
        p = j % 2
        pend.pop(j).wait()
        c1 = pltpu.async_copy(bufs[p], xs_hbm.at[idx1.at[j]], wsems[p])
        c2 = pltpu.async_copy(bufs[p], xs_hbm.at[idx2.at[j]], wsems[p])
        pw[j] = (c1, c2)
        if j + 2 < nsub:
            for c in pw.pop(j):
                c.wait()
            pend[j + 2] = read(j + 2)
    for cs in pw.values():
        for c in cs:
            c.wait()


def _sc_dispatch(hidden_states, s1, s2):
    s1r = s1.reshape(NW, TPW // DCH, DCH)
    s2r = s2.reshape(NW, TPW // DCH, DCH)
    k = functools.partial(
        pl.kernel,
        mesh=_sc_mesh(),
        out_type=jax.ShapeDtypeStruct((NPAD, H), jnp.float32),
        scratch_types=[
            pltpu.VMEM((DCH, H), jnp.float32),
            pltpu.VMEM((DCH, H), jnp.float32),
            pltpu.VMEM((TPW // DCH, DCH), jnp.int32),
            pltpu.VMEM((TPW // DCH, DCH), jnp.int32),
            pltpu.SemaphoreType.DMA,
            pltpu.SemaphoreType.DMA,
            pltpu.SemaphoreType.DMA,
            pltpu.SemaphoreType.DMA,
        ],
    )(_dispatch_body)
    return k(hidden_states, s1r, s2r)


_NSUB = TPW // CCH


def _combine_body(ys_hbm, s1_hbm, s2_hbm, rw1_hbm, rw2_hbm, out_hbm,
                  a0, a1, b0, b1, o0, o1, idx1, idx2, rw1v, rw2v,
                  ga0, ga1, gb0, gb1, os0, os1):
    wid = lax.axis_index("s") * NC + lax.axis_index("c")
    base = wid * TPW
    pltpu.sync_copy(s1_hbm.at[wid], idx1)
    pltpu.sync_copy(s2_hbm.at[wid], idx2)
    pltpu.sync_copy(rw1_hbm.at[wid], rw1v)
    pltpu.sync_copy(rw2_hbm.at[wid], rw2v)
    ab = (a0, a1)
    bb = (b0, b1)
    ob = (o0, o1)
    gas = (ga0, ga1)
    gbs = (gb0, gb1)
    oss = (os0, os1)

    def gather(j):
        p = j % 2
        c1 = pltpu.async_copy(ys_hbm.at[idx1.at[j]], ab[p], gas[p])
        c2 = pltpu.async_copy(ys_hbm.at[idx2.at[j]], bb[p], gbs[p])
        return c1, c2

    pend = {0: gather(0), 1: gather(1)}
    pstore = {}
    for j in range(_NSUB):
        p = j % 2
        c1, c2 = pend.pop(j)
        c1.wait()
        c2.wait()
        if j - 2 in pstore:
            pstore.pop(j - 2).wait()

        @pl.loop(0, CCH)
        def _(r):
            g = jnp.full((16,), j * CCH + r, jnp.int32)
            w1 = plsc.load_gather(rw1v, [g])
            w2 = plsc.load_gather(rw2v, [g])
            for c in range(0, H, 16):
                ob[p][r, pl.ds(c, 16)] = (ab[p][r, pl.ds(c, 16)] * w1
                                          + bb[p][r, pl.ds(c, 16)] * w2)

        pstore[j] = pltpu.async_copy(
            ob[p], out_hbm.at[pl.ds(base + j * CCH, CCH)], oss[p])
        if j + 2 < _NSUB:
            pend[j + 2] = gather(j + 2)
    for cp in pstore.values():
        cp.wait()


def _sc_combine(ys, s1, s2, rw1, rw2):
    s1r = s1.reshape(NW, _NSUB, CCH)
    s2r = s2.reshape(NW, _NSUB, CCH)
    rw1r = rw1.reshape(NW, TPW)
    rw2r = rw2.reshape(NW, TPW)
    k = functools.partial(
        pl.kernel,
        mesh=_sc_mesh(),
        out_type=jax.ShapeDtypeStruct((T, H), jnp.float32),
        scratch_types=(
            [pltpu.VMEM((CCH, H), jnp.float32) for _ in range(6)]
            + [pltpu.VMEM((_NSUB, CCH), jnp.int32) for _ in range(2)]
            + [pltpu.VMEM((TPW,), jnp.float32) for _ in range(2)]
            + [pltpu.SemaphoreType.DMA for _ in range(6)]
        ),
        compiler_params=_SC_PARAMS,
    )(_combine_body)
    return k(ys, s1r, s2r, rw1r, rw2r)


def _meta_body(lt_ref, s1_ref, s2_ref, rw1_ref, rw2_ref, be_ref, nbt_ref):
    lt = lt_ref[...]
    NEG = jnp.float32(-1e30)
    m1 = jnp.full((1, T), NEG, jnp.float32)
    i1 = jnp.zeros((1, T), jnp.int32)
    for e in range(E):
        ce = lt[e:e + 1, :]
        upd = ce > m1
        i1 = jnp.where(upd, e, i1)
        m1 = jnp.where(upd, ce, m1)
    m2 = jnp.full((1, T), NEG, jnp.float32)
    i2 = jnp.zeros((1, T), jnp.int32)
    for e in range(E):
        ce = jnp.where(i1 == e, NEG, lt[e:e + 1, :])
        upd = ce > m2
        i2 = jnp.where(upd, e, i2)
        m2 = jnp.where(upd, ce, m2)
    rw1_ref[...] = jax.nn.sigmoid(m1 - m2)
    rw2_ref[...] = jax.nn.sigmoid(m2 - m1)

    ids = jnp.concatenate([i1, i2], axis=1)
    eio = jax.lax.broadcasted_iota(jnp.int32, (E, K * T), 0)
    oh = (ids == eio).astype(jnp.float32)
    lio = jax.lax.broadcasted_iota(jnp.int32, (E, K * T), 1)
    c = oh
    sh = 1
    while sh < K * T:
        c = c + jnp.where(lio >= sh, pltpu.roll(c, sh, axis=1), 0.0)
        sh *= 2
    counts = c[:, K * T - 1:K * T]
    rank = jnp.sum(oh * (c - 1.0), axis=0, keepdims=True)
    nblk = jnp.ceil(counts / B)
    sio = jax.lax.broadcasted_iota(jnp.int32, (E, 1), 0)
    cb = nblk
    for s in (1, 2, 4):
        cb = cb + jnp.where(sio >= s, pltpu.roll(cb, s, axis=0), 0.0)
    row_off = (cb - nblk) * B
    slots = (rank + jnp.sum(oh * row_off, axis=0, keepdims=True)).astype(jnp.int32)
    s1_ref[...] = slots[:, :T]
    s2_ref[...] = slots[:, T:]
    bio = jax.lax.broadcasted_iota(jnp.int32, (E, NB), 1).astype(jnp.float32)
    becnt = jnp.sum((bio >= cb).astype(jnp.int32), axis=0, keepdims=True)
    last_e = jnp.max(jnp.where(nblk > 0.0, sio, 0), axis=0, keepdims=True)
    be_ref[...] = jnp.minimum(becnt, last_e)
    nbt_ref[...] = cb[E - 1:E, :].astype(jnp.int32)


def _tc_metadata(router_logits_t):
    out_shapes = [
        jax.ShapeDtypeStruct((1, T), jnp.int32),
        jax.ShapeDtypeStruct((1, T), jnp.int32),
        jax.ShapeDtypeStruct((1, T), jnp.float32),
        jax.ShapeDtypeStruct((1, T), jnp.float32),
        jax.ShapeDtypeStruct((1, NB), jnp.int32),
        jax.ShapeDtypeStruct((1, 1), jnp.int32),
    ]
    return pl.pallas_call(_meta_body, out_shape=out_shapes)(router_logits_t)


def _ffn_block(be_ref, nbt_ref, xs_ref, w13_ref, w2_ref, ys_ref,
               w13_bf, w2_bf):
    b = pl.program_id(0)

    @pl.when(b < nbt_ref[0])
    def _():
        prev = be_ref[jnp.maximum(b - 1, 0)]

        @pl.when((b == 0) | (be_ref[b] != prev))
        def _():
            w13_bf[...] = w13_ref[0].astype(jnp.bfloat16)
            w2_bf[...] = w2_ref[0].astype(jnp.bfloat16)

        x = xs_ref[...].astype(jnp.bfloat16)
        gu = jax.lax.dot_general(x, w13_bf[...], (((1,), (1,)), ((), ())),
                                 preferred_element_type=jnp.float32)
        g = gu[:, :I]
        u = gu[:, I:]
        act = (g * jax.nn.sigmoid(g) * u).astype(jnp.bfloat16)
        y = jax.lax.dot_general(act, w2_bf[...], (((1,), (1,)), ((), ())),
                                preferred_element_type=jnp.float32)
        ys_ref[...] = y


def _grouped_ffn(block_expert, nb_total, xs, w13, w2):
    grid_spec = pltpu.PrefetchScalarGridSpec(
        num_scalar_prefetch=2,
        grid=(NB,),
        in_specs=[
            pl.BlockSpec((B, H),
                         lambda b, be, nbt: (jnp.minimum(b, nbt[0] - 1), 0)),
            pl.BlockSpec((1, 2 * I, H), lambda b, be, nbt: (be[b], 0, 0)),
            pl.BlockSpec((1, H, I), lambda b, be, nbt: (be[b], 0, 0)),
        ],
        out_specs=pl.BlockSpec((B, H),
                               lambda b, be, nbt: (jnp.minimum(b, nbt[0] - 1), 0)),
        scratch_shapes=[
            pltpu.VMEM((2 * I, H), jnp.bfloat16),
            pltpu.VMEM((H, I), jnp.bfloat16),
        ],
    )
    return pl.pallas_call(
        _ffn_block,
        grid_spec=grid_spec,
        out_shape=jax.ShapeDtypeStruct((NPAD, H), jnp.float32),
        compiler_params=pltpu.CompilerParams(
            vmem_limit_bytes=64 * 1024 * 1024),
    )(block_expert, nb_total, xs, w13, w2)


def kernel(hidden_states, router_logits, w13, w2):
    s1o, s2o, rw1o, rw2o, beo, nbto = _tc_metadata(
        router_logits.astype(jnp.float32).T)
    s1 = s1o.reshape(T)
    s2 = s2o.reshape(T)
    rw1 = rw1o.reshape(T)
    rw2 = rw2o.reshape(T)
    block_expert = beo.reshape(NB)
    nb_total = nbto.reshape(1)

    xs = _sc_dispatch(hidden_states, s1, s2)

    ys = _grouped_ffn(block_expert, nb_total, xs, w13, w2)

    return _sc_combine(ys, s1, s2, rw1, rw2)
